# baseline (speedup 1.0000x reference)
import jax
import jax.numpy as jnp
from jax import lax
from jax.experimental import pallas as pl
from jax.experimental.pallas import tpu as pltpu

_K = 8


def _router_body(x_ref, w_ref, b_ref, scores_ref, idx_ref):
    x = x_ref[...]
    w = w_ref[...]
    logits = jnp.dot(x, w, preferred_element_type=jnp.float32) + b_ref[...]
    lt = logits.T

    n_exp, bm = lt.shape
    iota_e = lax.broadcasted_iota(jnp.int32, (n_exp, bm), 0).astype(jnp.float32)
    neg_inf = jnp.float32(-jnp.inf)

    cur = lt
    vals = []
    idxs = []
    for _ in range(_K):
        m = jnp.max(cur, axis=0, keepdims=True)
        at_max = cur == m
        idx = jnp.min(
            jnp.where(at_max, iota_e, jnp.float32(n_exp)), axis=0, keepdims=True
        )
        cur = jnp.where(iota_e == idx, neg_inf, cur)
        vals.append(m)
        idxs.append(idx)

    chosen = cur == neg_inf
    m0 = vals[0]
    denom = jnp.exp(vals[0] - m0)
    for v in vals[1:]:
        denom = denom + jnp.exp(v - m0)
    inv = 1.0 / denom
    scores_t = jnp.where(chosen, jnp.exp(lt - m0) * inv, jnp.float32(0.0))
    scores_ref[...] = scores_t.T

    idx_t = jnp.concatenate(idxs, axis=0)
    idx_ref[...] = idx_t.T.astype(jnp.int32)


def kernel(hidden_states, W, b):
    Bx, Sx, Hx = hidden_states.shape
    n_exp = W.shape[1]
    m_total = Bx * Sx
    x = hidden_states.reshape(m_total, Hx)
    b2 = b.reshape(1, n_exp)

    bm = 2048 if m_total % 2048 == 0 else m_total
    grid = (m_total // bm,)

    scores, indices = pl.pallas_call(
        _router_body,
        grid=grid,
        in_specs=[
            pl.BlockSpec((bm, Hx), lambda i: (i, 0)),
            pl.BlockSpec((Hx, n_exp), lambda i: (0, 0)),
            pl.BlockSpec((1, n_exp), lambda i: (0, 0)),
        ],
        out_specs=[
            pl.BlockSpec((bm, n_exp), lambda i: (i, 0)),
            pl.BlockSpec((bm, _K), lambda i: (i, 0)),
        ],
        out_shape=[
            jax.ShapeDtypeStruct((m_total, n_exp), jnp.float32),
            jax.ShapeDtypeStruct((m_total, _K), jnp.int32),
        ],
        compiler_params=pltpu.CompilerParams(
            dimension_semantics=("arbitrary",),
        ),
    )(x, W, b2)
    return scores, indices
